# baseline (speedup 1.0000x reference)
import functools

import jax
import jax.numpy as jnp
from jax import lax
from jax.experimental import pallas as pl
from jax.experimental.pallas import tpu as pltpu
from jax.experimental.pallas import tpu_sc as plsc

_EPS = 1e-5
_CH = 128
_NW = 32
_D = 64
_W = 128


def _relayout_pair(tview, vch, nch):
    F, D, V = tview.shape
    hch = vch // 2

    def body(in_ref, out_ref):
        xt = in_ref[0].T
        out_ref[0, :, 0:D] = xt[0:hch, :]
        out_ref[0, :, D:2 * D] = xt[hch:vch, :]

    out = pl.pallas_call(
        body,
        grid=(F, nch),
        in_specs=[pl.BlockSpec((1, D, vch), lambda f, c: (f, 0, c))],
        out_specs=pl.BlockSpec((1, hch, _W), lambda f, c: (f * nch + c, 0, 0)),
        out_shape=jax.ShapeDtypeStruct((F * nch, hch, _W), jnp.float32),
    )(tview)
    return out.reshape(F * nch * vch, _D)


def _remap_ids(ids, V, vch, nch):
    f = ids // V
    vl = ids - f * V
    c = vl // vch
    local = vl - c * vch
    hch = vch // 2
    return ((f * nch + c) * vch
            + jnp.where(local < hch, 2 * local, 2 * (local - hch) + 1))


def _sc_gather_item(table, ids, n_hist, n_pn):
    nch_h = n_hist // _CH
    nch_all = (n_hist + n_pn) // _CH
    iters = -(-nch_all // _NW)
    mesh = plsc.VectorSubcoreMesh(core_axis_name="c", subcore_axis_name="s")

    @functools.partial(
        pl.kernel,
        out_type=(jax.ShapeDtypeStruct((n_hist, _D), jnp.float32),
                  jax.ShapeDtypeStruct((n_pn, _D), jnp.float32)),
        mesh=mesh,
        scratch_types=[
            pltpu.VMEM((_CH,), jnp.int32),
            pltpu.VMEM((_CH, _D), jnp.float32),
            pltpu.SemaphoreType.DMA,
        ],
        compiler_params=pltpu.CompilerParams(use_tc_tiling_on_sc=False),
    )
    def gather(tbl, ids_hbm, out_h, out_pn, idx_v, rows_v, sem):
        wid = lax.axis_index("s") * 2 + lax.axis_index("c")

        def body(i, carry):
            c = wid + _NW * i

            @pl.when(c < nch_all)
            def _():
                pltpu.sync_copy(ids_hbm.at[pl.ds(c * _CH, _CH)], idx_v)
                pltpu.async_copy(tbl.at[idx_v], rows_v, sem).wait()

                @pl.when(c < nch_h)
                def _():
                    pltpu.sync_copy(rows_v, out_h.at[pl.ds(c * _CH, _CH)])

                @pl.when(c >= nch_h)
                def _():
                    pltpu.sync_copy(
                        rows_v, out_pn.at[pl.ds((c - nch_h) * _CH, _CH)])

            return carry

        lax.fori_loop(0, iters, body, 0)

    return gather(table, ids)


def _sc_gather(table, ids, n_out):
    nchunks = n_out // _CH
    iters = -(-nchunks // _NW)
    mesh = plsc.VectorSubcoreMesh(core_axis_name="c", subcore_axis_name="s")

    @functools.partial(
        pl.kernel,
        out_type=jax.ShapeDtypeStruct((n_out, _D), jnp.float32),
        mesh=mesh,
        scratch_types=[
            pltpu.VMEM((_CH,), jnp.int32),
            pltpu.VMEM((_CH, _D), jnp.float32),
            pltpu.SemaphoreType.DMA,
        ],
        compiler_params=pltpu.CompilerParams(use_tc_tiling_on_sc=False),
    )
    def gather(tbl, ids_hbm, out, idx_v, rows_v, sem):
        wid = lax.axis_index("s") * 2 + lax.axis_index("c")

        def body(i, carry):
            c = wid + _NW * i

            @pl.when(c < nchunks)
            def _():
                base = c * _CH
                pltpu.sync_copy(ids_hbm.at[pl.ds(base, _CH)], idx_v)
                pltpu.async_copy(tbl.at[idx_v], rows_v, sem).wait()
                pltpu.sync_copy(rows_v, out.at[pl.ds(base, _CH)])

            return carry

        lax.fori_loop(0, iters, body, 0)

    return gather(table, ids)


def _halfmask(shape):
    return lax.broadcasted_iota(jnp.int32, shape, len(shape) - 1) < _D


def _l2n_pair(x):
    m = _halfmask(x.shape)
    xe = jnp.where(m, x, 0.0)
    xo = jnp.where(m, 0.0, x)
    ne = jnp.sqrt(jnp.sum(xe * xe, axis=1, keepdims=True))
    no = jnp.sqrt(jnp.sum(xo * xo, axis=1, keepdims=True))
    rcp = jnp.where(m, 1.0 / jnp.maximum(ne, 1e-12),
                    1.0 / jnp.maximum(no, 1e-12))
    return x * rcp


def _att_body(hist_ref, pn_ref,
              wa1_ref, ba1_ref, ga1_ref, bea1_ref, ala1_ref,
              wa2_ref, ba2_ref,
              att_ref,
              tgt_scr, t13e_scr, t13o_scr,
              w23a_scr, w23b_scr, w4a_scr, w4b_scr,
              he_scr, ho_scr, stat_scr, bnc_scr):
    p = pl.program_id(0)
    l = pl.program_id(1)
    L, B2, W = hist_ref.shape
    cnt = float(2 * B2 * L)

    @pl.when((p == 0) & (l == 0))
    def _init():
        tgtp = _l2n_pair(pn_ref[0])
        tgt_scr[...] = tgtp
        z = jnp.zeros((_D, wa1_ref.shape[1]), jnp.float32)
        w13 = wa1_ref[0:_D, :] + wa1_ref[2 * _D:3 * _D, :]
        w23 = wa1_ref[_D:2 * _D, :] - wa1_ref[2 * _D:3 * _D, :]
        w4 = wa1_ref[3 * _D:4 * _D, :]
        w23a_scr[...] = jnp.concatenate([w23, z], axis=0)
        w23b_scr[...] = jnp.concatenate([z, w23], axis=0)
        w4a_scr[...] = jnp.concatenate([w4, z], axis=0)
        w4b_scr[...] = jnp.concatenate([z, w4], axis=0)
        w13a = jnp.concatenate([w13, z], axis=0)
        w13b = jnp.concatenate([z, w13], axis=0)
        t13e_scr[...] = (jnp.dot(tgtp, w13a,
                                 preferred_element_type=jnp.float32)
                         + ba1_ref[...])
        t13o_scr[...] = (jnp.dot(tgtp, w13b,
                                 preferred_element_type=jnp.float32)
                         + ba1_ref[...])
        stat_scr[...] = jnp.zeros_like(stat_scr)

    @pl.when(p == 0)
    def _phase0():
        hl = hist_ref[l]
        thl = tgt_scr[...] * hl
        he = (jnp.dot(hl, w23a_scr[...], preferred_element_type=jnp.float32)
              + jnp.dot(thl, w4a_scr[...],
                        preferred_element_type=jnp.float32)
              + t13e_scr[...])
        ho = (jnp.dot(hl, w23b_scr[...], preferred_element_type=jnp.float32)
              + jnp.dot(thl, w4b_scr[...],
                        preferred_element_type=jnp.float32)
              + t13o_scr[...])
        he_scr[l] = he
        ho_scr[l] = ho
        stat_scr[0:1, :] += (jnp.sum(he, axis=0, keepdims=True)
                             + jnp.sum(ho, axis=0, keepdims=True))
        stat_scr[1:2, :] += (jnp.sum(he * he, axis=0, keepdims=True)
                             + jnp.sum(ho * ho, axis=0, keepdims=True))

    @pl.when((p == 1) & (l == 0))
    def _stats():
        m = stat_scr[0:1, :] / cnt
        ex2 = stat_scr[1:2, :] / cnt
        v = ex2 - m * m
        rs = lax.rsqrt(v + _EPS)
        g = ga1_ref[...]
        v2 = g * g * v * (rs * rs)
        s2 = lax.rsqrt(v2 + _EPS)
        bnc_scr[0:1, :] = m
        bnc_scr[1:2, :] = rs
        bnc_scr[2:3, :] = g * s2
        att_ref[...] = jnp.zeros_like(att_ref)

    def _dice_w(h):
        xn = (h - bnc_scr[0:1, :]) * bnc_scr[1:2, :]
        bn = ga1_ref[...] * xn + bea1_ref[...]
        pgate = jax.nn.sigmoid(bnc_scr[2:3, :] * xn)
        al = ala1_ref[...]
        dice = bn * (al + pgate * (1.0 - al))
        return (jnp.sum(dice * wa2_ref[...], axis=1, keepdims=True)
                + ba2_ref[0, 0])

    @pl.when(p == 1)
    def _phase1():
        wle = _dice_w(he_scr[l])
        wlo = _dice_w(ho_scr[l])
        hl = hist_ref[l]
        att_ref[...] += jnp.where(_halfmask(hl.shape), wle, wlo) * hl


def _attention(hist3, pn3, wa1, ba1, ga1, bea1, ala1, wa2, ba2):
    L, B2, W = hist3.shape
    NA = wa1.shape[1]
    full = lambda a: pl.BlockSpec(a.shape, lambda p, l: (0,) * a.ndim)
    args = (hist3, pn3, wa1, ba1, ga1, bea1, ala1, wa2, ba2)
    return pl.pallas_call(
        _att_body,
        grid=(2, L),
        in_specs=[full(a) for a in args],
        out_specs=pl.BlockSpec((B2, W), lambda p, l: (0, 0)),
        out_shape=jax.ShapeDtypeStruct((B2, W), jnp.float32),
        scratch_shapes=[
            pltpu.VMEM((B2, W), jnp.float32),
            pltpu.VMEM((B2, NA), jnp.float32),
            pltpu.VMEM((B2, NA), jnp.float32),
            pltpu.VMEM((W, NA), jnp.float32),
            pltpu.VMEM((W, NA), jnp.float32),
            pltpu.VMEM((W, NA), jnp.float32),
            pltpu.VMEM((W, NA), jnp.float32),
            pltpu.VMEM((L, B2, NA), jnp.float32),
            pltpu.VMEM((L, B2, NA), jnp.float32),
            pltpu.VMEM((2, NA), jnp.float32),
            pltpu.VMEM((3, NA), jnp.float32),
        ],
        compiler_params=pltpu.CompilerParams(
            vmem_limit_bytes=63 * 1024 * 1024),
    )(*args)


def _bn_dice_pair(xe, xo, g, be, al):
    cnt = float(2 * xe.shape[0])
    m = (jnp.sum(xe, axis=0, keepdims=True)
         + jnp.sum(xo, axis=0, keepdims=True)) / cnt
    xce = xe - m
    xco = xo - m
    v = (jnp.sum(xce * xce, axis=0, keepdims=True)
         + jnp.sum(xco * xco, axis=0, keepdims=True)) / cnt
    rs = lax.rsqrt(v + _EPS)
    v2 = g * g * v * (rs * rs)
    s2 = lax.rsqrt(v2 + _EPS)
    gs2 = g * s2

    def dice(xc):
        xn = xc * rs
        bn = g * xn + be
        pgate = jax.nn.sigmoid(gs2 * xn)
        return bn * (al + pgate * (1.0 - al))

    return dice(xce), dice(xco)


def _tower_body(sp_ref, pn_ref, att_ref, psum_ref,
                wu1a_ref, wu1b_ref, bu1_ref, gu1_ref, beu1_ref, alu1_ref,
                wu2e_ref, wu2o_ref, bu2_ref, gu2_ref, beu2_ref, alu2_ref,
                y_ref):
    NS = sp_ref.shape[0]
    NNEG = pn_ref.shape[0] - 1
    B2 = att_ref.shape[0]
    NH = wu1a_ref.shape[2] // 2
    u = (jnp.dot(att_ref[...], wu1b_ref[...],
                 preferred_element_type=jnp.float32) + bu1_ref[...])
    for f in range(NS):
        u += jnp.dot(sp_ref[f], wu1a_ref[f],
                     preferred_element_type=jnp.float32)
    ue = u[:, 0:NH]
    uo = u[:, NH:2 * NH]
    de, do = _bn_dice_pair(ue, uo, gu1_ref[...], beu1_ref[...],
                           alu1_ref[...])
    u2 = (jnp.dot(de, wu2e_ref[...], preferred_element_type=jnp.float32)
          + jnp.dot(do, wu2o_ref[...], preferred_element_type=jnp.float32)
          + bu2_ref[...])
    cnt = float(2 * B2)
    m = jnp.dot(jnp.sum(u2, axis=0, keepdims=True), psum_ref[...],
                preferred_element_type=jnp.float32) / cnt
    xc = u2 - m
    v = jnp.dot(jnp.sum(xc * xc, axis=0, keepdims=True), psum_ref[...],
                preferred_element_type=jnp.float32) / cnt
    rs = lax.rsqrt(v + _EPS)
    g = gu2_ref[...]
    v2 = g * g * v * (rs * rs)
    s2 = lax.rsqrt(v2 + _EPS)
    xn = xc * rs
    bn = g * xn + beu2_ref[...]
    pgate = jax.nn.sigmoid(g * s2 * xn)
    al = alu2_ref[...]
    dice2 = bn * (al + pgate * (1.0 - al))
    user = _l2n_pair(dice2)
    hm = _halfmask(user.shape)
    for q in range(1 + NNEG):
        nen = _l2n_pair(pn_ref[q])
        prod = user * nen
        ye = jnp.sum(jnp.where(hm, prod, 0.0), axis=1, keepdims=True)
        yo = jnp.sum(jnp.where(hm, 0.0, prod), axis=1, keepdims=True)
        y_ref[:, q:q + 1] = ye
        y_ref[:, 1 + NNEG + q:2 + NNEG + q] = yo


def _tower(sp3, pn3, att, psum, wu1a, wu1b, bu1, gu1, beu1, alu1,
           wu2e, wu2o, bu2, gu2, beu2, alu2):
    B2 = att.shape[0]
    NNEG = pn3.shape[0] - 1
    full = lambda a: pl.BlockSpec(a.shape, lambda: (0,) * a.ndim)
    args = (sp3, pn3, att, psum, wu1a, wu1b, bu1, gu1, beu1, alu1,
            wu2e, wu2o, bu2, gu2, beu2, alu2)
    return pl.pallas_call(
        _tower_body,
        in_specs=[full(a) for a in args],
        out_specs=pl.BlockSpec((B2, 2 * (1 + NNEG)), lambda: (0, 0)),
        out_shape=jax.ShapeDtypeStruct((B2, 2 * (1 + NNEG)), jnp.float32),
        compiler_params=pltpu.CompilerParams(
            vmem_limit_bytes=63 * 1024 * 1024),
    )(*args)


def kernel(sparse_ids, hist_ids, pos_ids, neg_ids, table_sparse, table_item,
           W_a1, b_a1, g_a1, be_a1, al_a1, W_a2, b_a2,
           W_u1, b_u1, g_u1, be_u1, al_u1,
           W_u2, b_u2, g_u2, be_u2, al_u2):
    B, NS = sparse_ids.shape
    L = hist_ids.shape[1]
    NNEG = neg_ids.shape[1]
    VS = table_sparse.shape[1]
    D = table_item.shape[1]
    B2 = B // 2

    ti_pair = _relayout_pair(table_item.T[None], 6400, 16)
    ts_view = jnp.transpose(table_sparse, (0, 2, 1))
    ts_view = lax.optimization_barrier((ts_view, ti_pair))[0]
    ts_pair = _relayout_pair(ts_view, 2560, 4)

    ids_item = _remap_ids(jnp.concatenate([
        hist_ids.astype(jnp.int32).T.reshape(-1),
        pos_ids.astype(jnp.int32).reshape(-1),
        neg_ids.astype(jnp.int32).T.reshape(-1),
    ]), table_item.shape[0], 6400, 16)
    ids_sp = _remap_ids(
        (sparse_ids.astype(jnp.int32).T
         + (jnp.arange(NS, dtype=jnp.int32) * VS)[:, None]).reshape(-1),
        VS, 2560, 4)

    out_hist, out_pn = _sc_gather_item(ti_pair, ids_item, B * L,
                                       B * (1 + NNEG))
    out_sp = _sc_gather(ts_pair, ids_sp, B * NS)
    hist3 = out_hist.reshape(L, B2, _W)
    pn3 = out_pn.reshape(1 + NNEG, B2, _W)
    sp3 = out_sp.reshape(NS, B2, _W)

    NH = W_u1.shape[1]
    wu1a3 = W_u1[:NS * D].reshape(NS, D, NH)
    wu1a = (jnp.pad(wu1a3, ((0, 0), (0, D), (0, NH)))
            + jnp.pad(wu1a3, ((0, 0), (D, 0), (NH, 0))))
    wu1b = (jnp.pad(W_u1[NS * D:], ((0, D), (0, NH)))
            + jnp.pad(W_u1[NS * D:], ((D, 0), (NH, 0))))
    wu2e = jnp.pad(W_u2, ((0, 0), (0, D)))
    wu2o = jnp.pad(W_u2, ((0, 0), (D, 0)))
    psum = jnp.tile(jnp.eye(D, dtype=jnp.float32), (2, 2))
    row = lambda a: a.reshape(1, -1)
    pair = lambda a: jnp.tile(a, 2).reshape(1, -1)

    att = _attention(hist3, pn3, W_a1, row(b_a1), row(g_a1), row(be_a1),
                     row(al_a1), W_a2.reshape(1, -1), b_a2.reshape(1, 1))
    ypair = _tower(sp3, pn3, att, psum, wu1a, wu1b,
                   pair(b_u1), row(g_u1), row(be_u1), row(al_u1),
                   wu2e, wu2o, pair(b_u2), pair(g_u2), pair(be_u2),
                   pair(al_u2))
    return ypair.reshape(B, 1 + NNEG)


# speedup vs baseline: 1.2176x; 1.2176x over previous
import functools

import jax
import jax.numpy as jnp
from jax import lax
from jax.experimental import pallas as pl
from jax.experimental.pallas import tpu as pltpu
from jax.experimental.pallas import tpu_sc as plsc

_EPS = 1e-5
_CH = 128
_NW = 32
_D = 64
_W = 128


def _relayout_pair(tview, vch, nch):
    F, D, V = tview.shape
    hch = vch // 2

    def body(in_ref, out_ref):
        xt = in_ref[0].T
        out_ref[0, :, 0:D] = xt[0:hch, :]
        out_ref[0, :, D:2 * D] = xt[hch:vch, :]

    out = pl.pallas_call(
        body,
        grid=(F, nch),
        in_specs=[pl.BlockSpec((1, D, vch), lambda f, c: (f, 0, c))],
        out_specs=pl.BlockSpec((1, hch, _W), lambda f, c: (f * nch + c, 0, 0)),
        out_shape=jax.ShapeDtypeStruct((F * nch, hch, _W), jnp.float32),
    )(tview)
    return out.reshape(F * nch * vch, _D)


def _remap_ids(ids, V, vch, nch):
    f = ids // V
    vl = ids - f * V
    c = vl // vch
    local = vl - c * vch
    hch = vch // 2
    return ((f * nch + c) * vch
            + jnp.where(local < hch, 2 * local, 2 * (local - hch) + 1))


def _sc_gather_item(table, ids, n_hist, n_pn):
    nch_h = n_hist // _CH
    nch_all = (n_hist + n_pn) // _CH
    iters = -(-nch_all // _NW)
    mesh = plsc.VectorSubcoreMesh(core_axis_name="c", subcore_axis_name="s")

    @functools.partial(
        pl.kernel,
        out_type=(jax.ShapeDtypeStruct((n_hist, _D), jnp.float32),
                  jax.ShapeDtypeStruct((n_pn, _D), jnp.float32)),
        mesh=mesh,
        scratch_types=[
            pltpu.VMEM((_CH,), jnp.int32),
            pltpu.VMEM((_CH, _D), jnp.float32),
            pltpu.SemaphoreType.DMA,
        ],
        compiler_params=pltpu.CompilerParams(use_tc_tiling_on_sc=False),
    )
    def gather(tbl, ids_hbm, out_h, out_pn, idx_v, rows_v, sem):
        wid = lax.axis_index("s") * 2 + lax.axis_index("c")

        def body(i, carry):
            c = wid + _NW * i

            @pl.when(c < nch_all)
            def _():
                pltpu.sync_copy(ids_hbm.at[pl.ds(c * _CH, _CH)], idx_v)
                pltpu.async_copy(tbl.at[idx_v], rows_v, sem).wait()

                @pl.when(c < nch_h)
                def _():
                    pltpu.sync_copy(rows_v, out_h.at[pl.ds(c * _CH, _CH)])

                @pl.when(c >= nch_h)
                def _():
                    pltpu.sync_copy(
                        rows_v, out_pn.at[pl.ds((c - nch_h) * _CH, _CH)])

            return carry

        lax.fori_loop(0, iters, body, 0)

    return gather(table, ids)


def _sc_gather(table, ids, n_out):
    nchunks = n_out // _CH
    iters = -(-nchunks // _NW)
    mesh = plsc.VectorSubcoreMesh(core_axis_name="c", subcore_axis_name="s")

    @functools.partial(
        pl.kernel,
        out_type=jax.ShapeDtypeStruct((n_out, _D), jnp.float32),
        mesh=mesh,
        scratch_types=[
            pltpu.VMEM((_CH,), jnp.int32),
            pltpu.VMEM((_CH, _D), jnp.float32),
            pltpu.SemaphoreType.DMA,
        ],
        compiler_params=pltpu.CompilerParams(use_tc_tiling_on_sc=False),
    )
    def gather(tbl, ids_hbm, out, idx_v, rows_v, sem):
        wid = lax.axis_index("s") * 2 + lax.axis_index("c")

        def body(i, carry):
            c = wid + _NW * i

            @pl.when(c < nchunks)
            def _():
                base = c * _CH
                pltpu.sync_copy(ids_hbm.at[pl.ds(base, _CH)], idx_v)
                pltpu.async_copy(tbl.at[idx_v], rows_v, sem).wait()
                pltpu.sync_copy(rows_v, out.at[pl.ds(base, _CH)])

            return carry

        lax.fori_loop(0, iters, body, 0)

    return gather(table, ids)


def _halfmask(shape):
    return lax.broadcasted_iota(jnp.int32, shape, len(shape) - 1) < _D


def _l2n_pair(x):
    m = _halfmask(x.shape)
    xe = jnp.where(m, x, 0.0)
    xo = jnp.where(m, 0.0, x)
    ne = jnp.sqrt(jnp.sum(xe * xe, axis=1, keepdims=True))
    no = jnp.sqrt(jnp.sum(xo * xo, axis=1, keepdims=True))
    rcp = jnp.where(m, 1.0 / jnp.maximum(ne, 1e-12),
                    1.0 / jnp.maximum(no, 1e-12))
    return x * rcp


def _att_body(hist_ref, pn_ref,
              wa1_ref, ba1_ref, ga1_ref, bea1_ref, ala1_ref,
              wa2_ref, ba2_ref,
              att_ref,
              tgt_scr, t13e_scr, t13o_scr,
              w23a_scr, w23b_scr, w4a_scr, w4b_scr,
              he_scr, ho_scr, stat_scr, bnc_scr):
    p = pl.program_id(0)
    l = pl.program_id(1)
    L, B2, W = hist_ref.shape
    cnt = float(2 * B2 * L)

    @pl.when((p == 0) & (l == 0))
    def _init():
        tgtp = _l2n_pair(pn_ref[0])
        tgt_scr[...] = tgtp
        z = jnp.zeros((_D, wa1_ref.shape[1]), jnp.float32)
        w13 = wa1_ref[0:_D, :] + wa1_ref[2 * _D:3 * _D, :]
        w23 = wa1_ref[_D:2 * _D, :] - wa1_ref[2 * _D:3 * _D, :]
        w4 = wa1_ref[3 * _D:4 * _D, :]
        w23a_scr[...] = jnp.concatenate([w23, z], axis=0)
        w23b_scr[...] = jnp.concatenate([z, w23], axis=0)
        w4a_scr[...] = jnp.concatenate([w4, z], axis=0)
        w4b_scr[...] = jnp.concatenate([z, w4], axis=0)
        w13a = jnp.concatenate([w13, z], axis=0)
        w13b = jnp.concatenate([z, w13], axis=0)
        t13e_scr[...] = (jnp.dot(tgtp, w13a,
                                 preferred_element_type=jnp.float32)
                         + ba1_ref[...])
        t13o_scr[...] = (jnp.dot(tgtp, w13b,
                                 preferred_element_type=jnp.float32)
                         + ba1_ref[...])
        stat_scr[...] = jnp.zeros_like(stat_scr)

    @pl.when(p == 0)
    def _phase0():
        hl = hist_ref[l]
        thl = tgt_scr[...] * hl
        he = (jnp.dot(hl, w23a_scr[...], preferred_element_type=jnp.float32)
              + jnp.dot(thl, w4a_scr[...],
                        preferred_element_type=jnp.float32)
              + t13e_scr[...])
        ho = (jnp.dot(hl, w23b_scr[...], preferred_element_type=jnp.float32)
              + jnp.dot(thl, w4b_scr[...],
                        preferred_element_type=jnp.float32)
              + t13o_scr[...])
        he_scr[l] = he
        ho_scr[l] = ho
        stat_scr[0:1, :] += (jnp.sum(he, axis=0, keepdims=True)
                             + jnp.sum(ho, axis=0, keepdims=True))
        stat_scr[1:2, :] += (jnp.sum(he * he, axis=0, keepdims=True)
                             + jnp.sum(ho * ho, axis=0, keepdims=True))

    @pl.when((p == 1) & (l == 0))
    def _stats():
        m = stat_scr[0:1, :] / cnt
        ex2 = stat_scr[1:2, :] / cnt
        v = ex2 - m * m
        rs = lax.rsqrt(v + _EPS)
        g = ga1_ref[...]
        v2 = g * g * v * (rs * rs)
        s2 = lax.rsqrt(v2 + _EPS)
        bnc_scr[0:1, :] = m
        bnc_scr[1:2, :] = rs
        bnc_scr[2:3, :] = g * s2
        att_ref[...] = jnp.zeros_like(att_ref)

    def _dice_w(h):
        xn = (h - bnc_scr[0:1, :]) * bnc_scr[1:2, :]
        bn = ga1_ref[...] * xn + bea1_ref[...]
        pgate = jax.nn.sigmoid(bnc_scr[2:3, :] * xn)
        al = ala1_ref[...]
        dice = bn * (al + pgate * (1.0 - al))
        return (jnp.sum(dice * wa2_ref[...], axis=1, keepdims=True)
                + ba2_ref[0, 0])

    @pl.when(p == 1)
    def _phase1():
        wle = _dice_w(he_scr[l])
        wlo = _dice_w(ho_scr[l])
        hl = hist_ref[l]
        att_ref[...] += jnp.where(_halfmask(hl.shape), wle, wlo) * hl


def _attention(hist3, pn3, wa1, ba1, ga1, bea1, ala1, wa2, ba2):
    L, B2, W = hist3.shape
    NA = wa1.shape[1]
    full = lambda a: pl.BlockSpec(a.shape, lambda p, l: (0,) * a.ndim)
    args = (hist3, pn3, wa1, ba1, ga1, bea1, ala1, wa2, ba2)
    return pl.pallas_call(
        _att_body,
        grid=(2, L),
        in_specs=[full(a) for a in args],
        out_specs=pl.BlockSpec((B2, W), lambda p, l: (0, 0)),
        out_shape=jax.ShapeDtypeStruct((B2, W), jnp.float32),
        scratch_shapes=[
            pltpu.VMEM((B2, W), jnp.float32),
            pltpu.VMEM((B2, NA), jnp.float32),
            pltpu.VMEM((B2, NA), jnp.float32),
            pltpu.VMEM((W, NA), jnp.float32),
            pltpu.VMEM((W, NA), jnp.float32),
            pltpu.VMEM((W, NA), jnp.float32),
            pltpu.VMEM((W, NA), jnp.float32),
            pltpu.VMEM((L, B2, NA), jnp.float32),
            pltpu.VMEM((L, B2, NA), jnp.float32),
            pltpu.VMEM((2, NA), jnp.float32),
            pltpu.VMEM((3, NA), jnp.float32),
        ],
        compiler_params=pltpu.CompilerParams(
            vmem_limit_bytes=63 * 1024 * 1024),
    )(*args)


def _bn_dice_pair(xe, xo, g, be, al):
    cnt = float(2 * xe.shape[0])
    m = (jnp.sum(xe, axis=0, keepdims=True)
         + jnp.sum(xo, axis=0, keepdims=True)) / cnt
    xce = xe - m
    xco = xo - m
    v = (jnp.sum(xce * xce, axis=0, keepdims=True)
         + jnp.sum(xco * xco, axis=0, keepdims=True)) / cnt
    rs = lax.rsqrt(v + _EPS)
    v2 = g * g * v * (rs * rs)
    s2 = lax.rsqrt(v2 + _EPS)
    gs2 = g * s2

    def dice(xc):
        xn = xc * rs
        bn = g * xn + be
        pgate = jax.nn.sigmoid(gs2 * xn)
        return bn * (al + pgate * (1.0 - al))

    return dice(xce), dice(xco)


def _tower_body(sp_ref, pn_ref, att_ref, psum_ref,
                wu1a_ref, wu1b_ref, bu1_ref, gu1_ref, beu1_ref, alu1_ref,
                wu2e_ref, wu2o_ref, bu2_ref, gu2_ref, beu2_ref, alu2_ref,
                y_ref):
    NS = sp_ref.shape[0]
    NNEG = pn_ref.shape[0] - 1
    B2 = att_ref.shape[0]
    NH = wu1a_ref.shape[2] // 2
    u = (jnp.dot(att_ref[...], wu1b_ref[...],
                 preferred_element_type=jnp.float32) + bu1_ref[...])
    for f in range(NS):
        u += jnp.dot(sp_ref[f], wu1a_ref[f],
                     preferred_element_type=jnp.float32)
    ue = u[:, 0:NH]
    uo = u[:, NH:2 * NH]
    de, do = _bn_dice_pair(ue, uo, gu1_ref[...], beu1_ref[...],
                           alu1_ref[...])
    u2 = (jnp.dot(de, wu2e_ref[...], preferred_element_type=jnp.float32)
          + jnp.dot(do, wu2o_ref[...], preferred_element_type=jnp.float32)
          + bu2_ref[...])
    cnt = float(2 * B2)
    m = jnp.dot(jnp.sum(u2, axis=0, keepdims=True), psum_ref[...],
                preferred_element_type=jnp.float32) / cnt
    xc = u2 - m
    v = jnp.dot(jnp.sum(xc * xc, axis=0, keepdims=True), psum_ref[...],
                preferred_element_type=jnp.float32) / cnt
    rs = lax.rsqrt(v + _EPS)
    g = gu2_ref[...]
    v2 = g * g * v * (rs * rs)
    s2 = lax.rsqrt(v2 + _EPS)
    xn = xc * rs
    bn = g * xn + beu2_ref[...]
    pgate = jax.nn.sigmoid(g * s2 * xn)
    al = alu2_ref[...]
    dice2 = bn * (al + pgate * (1.0 - al))
    user = _l2n_pair(dice2)
    hm = _halfmask(user.shape)
    for q in range(1 + NNEG):
        nen = _l2n_pair(pn_ref[q])
        prod = user * nen
        ye = jnp.sum(jnp.where(hm, prod, 0.0), axis=1, keepdims=True)
        yo = jnp.sum(jnp.where(hm, 0.0, prod), axis=1, keepdims=True)
        y_ref[:, q:q + 1] = ye
        y_ref[:, 1 + NNEG + q:2 + NNEG + q] = yo


def _tower(sp3, pn3, att, psum, wu1a, wu1b, bu1, gu1, beu1, alu1,
           wu2e, wu2o, bu2, gu2, beu2, alu2):
    B2 = att.shape[0]
    NNEG = pn3.shape[0] - 1
    full = lambda a: pl.BlockSpec(a.shape, lambda: (0,) * a.ndim)
    args = (sp3, pn3, att, psum, wu1a, wu1b, bu1, gu1, beu1, alu1,
            wu2e, wu2o, bu2, gu2, beu2, alu2)
    return pl.pallas_call(
        _tower_body,
        in_specs=[full(a) for a in args],
        out_specs=pl.BlockSpec((B2, 2 * (1 + NNEG)), lambda: (0, 0)),
        out_shape=jax.ShapeDtypeStruct((B2, 2 * (1 + NNEG)), jnp.float32),
        compiler_params=pltpu.CompilerParams(
            vmem_limit_bytes=63 * 1024 * 1024),
    )(*args)


def kernel(sparse_ids, hist_ids, pos_ids, neg_ids, table_sparse, table_item,
           W_a1, b_a1, g_a1, be_a1, al_a1, W_a2, b_a2,
           W_u1, b_u1, g_u1, be_u1, al_u1,
           W_u2, b_u2, g_u2, be_u2, al_u2):
    B, NS = sparse_ids.shape
    L = hist_ids.shape[1]
    NNEG = neg_ids.shape[1]
    VS = table_sparse.shape[1]
    D = table_item.shape[1]
    B2 = B // 2

    ti_pair = _relayout_pair(table_item.T[None], 6400, 16)
    ts_view = jnp.transpose(table_sparse, (0, 2, 1))
    ts_view = lax.optimization_barrier((ts_view, ti_pair))[0]
    ts_pair = _relayout_pair(ts_view, VS, 1)

    ids_item = _remap_ids(jnp.concatenate([
        hist_ids.astype(jnp.int32).T.reshape(-1),
        pos_ids.astype(jnp.int32).reshape(-1),
        neg_ids.astype(jnp.int32).T.reshape(-1),
    ]), table_item.shape[0], 6400, 16)
    ids_sp = _remap_ids(
        (sparse_ids.astype(jnp.int32).T
         + (jnp.arange(NS, dtype=jnp.int32) * VS)[:, None]).reshape(-1),
        VS, VS, 1)

    out_hist, out_pn = _sc_gather_item(ti_pair, ids_item, B * L,
                                       B * (1 + NNEG))
    out_sp = _sc_gather(ts_pair, ids_sp, B * NS)
    hist3 = out_hist.reshape(L, B2, _W)
    pn3 = out_pn.reshape(1 + NNEG, B2, _W)
    sp3 = out_sp.reshape(NS, B2, _W)

    NH = W_u1.shape[1]
    wu1a3 = W_u1[:NS * D].reshape(NS, D, NH)
    wu1a = (jnp.pad(wu1a3, ((0, 0), (0, D), (0, NH)))
            + jnp.pad(wu1a3, ((0, 0), (D, 0), (NH, 0))))
    wu1b = (jnp.pad(W_u1[NS * D:], ((0, D), (0, NH)))
            + jnp.pad(W_u1[NS * D:], ((D, 0), (NH, 0))))
    wu2e = jnp.pad(W_u2, ((0, 0), (0, D)))
    wu2o = jnp.pad(W_u2, ((0, 0), (D, 0)))
    psum = jnp.tile(jnp.eye(D, dtype=jnp.float32), (2, 2))
    row = lambda a: a.reshape(1, -1)
    pair = lambda a: jnp.tile(a, 2).reshape(1, -1)

    att = _attention(hist3, pn3, W_a1, row(b_a1), row(g_a1), row(be_a1),
                     row(al_a1), W_a2.reshape(1, -1), b_a2.reshape(1, 1))
    ypair = _tower(sp3, pn3, att, psum, wu1a, wu1b,
                   pair(b_u1), row(g_u1), row(be_u1), row(al_u1),
                   wu2e, wu2o, pair(b_u2), pair(g_u2), pair(be_u2),
                   pair(al_u2))
    return ypair.reshape(B, 1 + NNEG)


# speedup vs baseline: 1.2693x; 1.0425x over previous
import functools

import jax
import jax.numpy as jnp
from jax import lax
from jax.experimental import pallas as pl
from jax.experimental.pallas import tpu as pltpu
from jax.experimental.pallas import tpu_sc as plsc

_EPS = 1e-5
_CH = 128
_NW = 32
_D = 64
_W = 128


def _relayout_pair(tview, vch, nch):
    F, D, V = tview.shape
    hch = vch // 2

    def body(in_ref, out_ref):
        xt = in_ref[0].T
        out_ref[0, :, 0:D] = xt[0:hch, :]
        out_ref[0, :, D:2 * D] = xt[hch:vch, :]

    out = pl.pallas_call(
        body,
        grid=(F, nch),
        in_specs=[pl.BlockSpec((1, D, vch), lambda f, c: (f, 0, c))],
        out_specs=pl.BlockSpec((1, hch, _W), lambda f, c: (f * nch + c, 0, 0)),
        out_shape=jax.ShapeDtypeStruct((F * nch, hch, _W), jnp.float32),
    )(tview)
    return out.reshape(F * nch * vch, _D)


def _remap_ids(ids, V, vch, nch):
    f = ids // V
    vl = ids - f * V
    c = vl // vch
    local = vl - c * vch
    hch = vch // 2
    return ((f * nch + c) * vch
            + jnp.where(local < hch, 2 * local, 2 * (local - hch) + 1))


def _sc_gather_item(table, ids, n_hist, n_pn):
    nch_h = n_hist // _CH
    nch_all = (n_hist + n_pn) // _CH
    iters = -(-nch_all // _NW)
    mesh = plsc.VectorSubcoreMesh(core_axis_name="c", subcore_axis_name="s")

    @functools.partial(
        pl.kernel,
        out_type=(jax.ShapeDtypeStruct((n_hist, _D), jnp.float32),
                  jax.ShapeDtypeStruct((n_pn, _D), jnp.float32)),
        mesh=mesh,
        scratch_types=[
            pltpu.VMEM((_CH,), jnp.int32),
            pltpu.VMEM((_CH, _D), jnp.float32),
            pltpu.SemaphoreType.DMA,
        ],
        compiler_params=pltpu.CompilerParams(use_tc_tiling_on_sc=False),
    )
    def gather(tbl, ids_hbm, out_h, out_pn, idx_v, rows_v, sem):
        wid = lax.axis_index("s") * 2 + lax.axis_index("c")

        def body(i, carry):
            c = wid + _NW * i

            @pl.when(c < nch_all)
            def _():
                pltpu.sync_copy(ids_hbm.at[pl.ds(c * _CH, _CH)], idx_v)
                pltpu.async_copy(tbl.at[idx_v], rows_v, sem).wait()

                @pl.when(c < nch_h)
                def _():
                    pltpu.sync_copy(rows_v, out_h.at[pl.ds(c * _CH, _CH)])

                @pl.when(c >= nch_h)
                def _():
                    pltpu.sync_copy(
                        rows_v, out_pn.at[pl.ds((c - nch_h) * _CH, _CH)])

            return carry

        lax.fori_loop(0, iters, body, 0)

    return gather(table, ids)


def _sc_gather(table, ids, n_out):
    nchunks = n_out // _CH
    iters = -(-nchunks // _NW)
    mesh = plsc.VectorSubcoreMesh(core_axis_name="c", subcore_axis_name="s")

    @functools.partial(
        pl.kernel,
        out_type=jax.ShapeDtypeStruct((n_out, _D), jnp.float32),
        mesh=mesh,
        scratch_types=[
            pltpu.VMEM((_CH,), jnp.int32),
            pltpu.VMEM((_CH, _D), jnp.float32),
            pltpu.SemaphoreType.DMA,
        ],
        compiler_params=pltpu.CompilerParams(use_tc_tiling_on_sc=False),
    )
    def gather(tbl, ids_hbm, out, idx_v, rows_v, sem):
        wid = lax.axis_index("s") * 2 + lax.axis_index("c")

        def body(i, carry):
            c = wid + _NW * i

            @pl.when(c < nchunks)
            def _():
                base = c * _CH
                pltpu.sync_copy(ids_hbm.at[pl.ds(base, _CH)], idx_v)
                pltpu.async_copy(tbl.at[idx_v], rows_v, sem).wait()
                pltpu.sync_copy(rows_v, out.at[pl.ds(base, _CH)])

            return carry

        lax.fori_loop(0, iters, body, 0)

    return gather(table, ids)


def _halfmask(shape):
    return lax.broadcasted_iota(jnp.int32, shape, len(shape) - 1) < _D


def _l2n_pair(x):
    m = _halfmask(x.shape)
    xe = jnp.where(m, x, 0.0)
    xo = jnp.where(m, 0.0, x)
    ne = jnp.sqrt(jnp.sum(xe * xe, axis=1, keepdims=True))
    no = jnp.sqrt(jnp.sum(xo * xo, axis=1, keepdims=True))
    rcp = jnp.where(m, 1.0 / jnp.maximum(ne, 1e-12),
                    1.0 / jnp.maximum(no, 1e-12))
    return x * rcp


def _att_body(hist_ref, pn_ref,
              wa1_ref, ba1_ref, ga1_ref, bea1_ref, ala1_ref,
              wa2_ref, ba2_ref,
              att_ref,
              tgt_scr, t13e_scr, t13o_scr,
              w23a_scr, w23b_scr, w4a_scr, w4b_scr,
              he_scr, ho_scr, stat_scr, bnc_scr):
    p = pl.program_id(0)
    l = pl.program_id(1)
    L2, R, W = hist_ref.shape
    B2 = R // 2
    cnt = float(2 * R * L2)

    @pl.when((p == 0) & (l == 0))
    def _init():
        tgtp = _l2n_pair(pn_ref[0])
        tgt_scr[0:B2, :] = tgtp
        tgt_scr[B2:R, :] = tgtp
        z = jnp.zeros((_D, wa1_ref.shape[1]), jnp.float32)
        w13 = wa1_ref[0:_D, :] + wa1_ref[2 * _D:3 * _D, :]
        w23 = wa1_ref[_D:2 * _D, :] - wa1_ref[2 * _D:3 * _D, :]
        w4 = wa1_ref[3 * _D:4 * _D, :]
        w23a_scr[...] = jnp.concatenate([w23, z], axis=0)
        w23b_scr[...] = jnp.concatenate([z, w23], axis=0)
        w4a_scr[...] = jnp.concatenate([w4, z], axis=0)
        w4b_scr[...] = jnp.concatenate([z, w4], axis=0)
        w13a = jnp.concatenate([w13, z], axis=0)
        w13b = jnp.concatenate([z, w13], axis=0)
        te = (jnp.dot(tgtp, w13a, preferred_element_type=jnp.float32)
              + ba1_ref[...])
        to = (jnp.dot(tgtp, w13b, preferred_element_type=jnp.float32)
              + ba1_ref[...])
        t13e_scr[0:B2, :] = te
        t13e_scr[B2:R, :] = te
        t13o_scr[0:B2, :] = to
        t13o_scr[B2:R, :] = to
        stat_scr[...] = jnp.zeros_like(stat_scr)

    @pl.when(p == 0)
    def _phase0():
        hl = hist_ref[l]
        thl = tgt_scr[...] * hl
        he = (jnp.dot(hl, w23a_scr[...], preferred_element_type=jnp.float32)
              + jnp.dot(thl, w4a_scr[...],
                        preferred_element_type=jnp.float32)
              + t13e_scr[...])
        ho = (jnp.dot(hl, w23b_scr[...], preferred_element_type=jnp.float32)
              + jnp.dot(thl, w4b_scr[...],
                        preferred_element_type=jnp.float32)
              + t13o_scr[...])
        he_scr[l] = he
        ho_scr[l] = ho
        stat_scr[0:1, :] += (jnp.sum(he, axis=0, keepdims=True)
                             + jnp.sum(ho, axis=0, keepdims=True))
        stat_scr[1:2, :] += (jnp.sum(he * he, axis=0, keepdims=True)
                             + jnp.sum(ho * ho, axis=0, keepdims=True))

    @pl.when((p == 1) & (l == 0))
    def _stats():
        m = stat_scr[0:1, :] / cnt
        ex2 = stat_scr[1:2, :] / cnt
        v = ex2 - m * m
        rs = lax.rsqrt(v + _EPS)
        g = ga1_ref[...]
        v2 = g * g * v * (rs * rs)
        s2 = lax.rsqrt(v2 + _EPS)
        bnc_scr[0:1, :] = m
        bnc_scr[1:2, :] = rs
        bnc_scr[2:3, :] = g * s2
        att_ref[...] = jnp.zeros_like(att_ref)

    def _dice_w(h):
        xn = (h - bnc_scr[0:1, :]) * bnc_scr[1:2, :]
        bn = ga1_ref[...] * xn + bea1_ref[...]
        pgate = jax.nn.sigmoid(bnc_scr[2:3, :] * xn)
        al = ala1_ref[...]
        dice = bn * (al + pgate * (1.0 - al))
        return (jnp.sum(dice * wa2_ref[...], axis=1, keepdims=True)
                + ba2_ref[0, 0])

    @pl.when(p == 1)
    def _phase1():
        wle = _dice_w(he_scr[l])
        wlo = _dice_w(ho_scr[l])
        hl = hist_ref[l]
        v = jnp.where(_halfmask(hl.shape), wle, wlo) * hl
        att_ref[...] += v[0:B2, :] + v[B2:R, :]


def _attention(hist3, pn3, wa1, ba1, ga1, bea1, ala1, wa2, ba2):
    L2, R, W = hist3.shape
    B2 = R // 2
    NA = wa1.shape[1]
    full = lambda a: pl.BlockSpec(a.shape, lambda p, l: (0,) * a.ndim)
    args = (hist3, pn3, wa1, ba1, ga1, bea1, ala1, wa2, ba2)
    return pl.pallas_call(
        _att_body,
        grid=(2, L2),
        in_specs=[full(a) for a in args],
        out_specs=pl.BlockSpec((B2, W), lambda p, l: (0, 0)),
        out_shape=jax.ShapeDtypeStruct((B2, W), jnp.float32),
        scratch_shapes=[
            pltpu.VMEM((R, W), jnp.float32),
            pltpu.VMEM((R, NA), jnp.float32),
            pltpu.VMEM((R, NA), jnp.float32),
            pltpu.VMEM((W, NA), jnp.float32),
            pltpu.VMEM((W, NA), jnp.float32),
            pltpu.VMEM((W, NA), jnp.float32),
            pltpu.VMEM((W, NA), jnp.float32),
            pltpu.VMEM((L2, R, NA), jnp.float32),
            pltpu.VMEM((L2, R, NA), jnp.float32),
            pltpu.VMEM((2, NA), jnp.float32),
            pltpu.VMEM((3, NA), jnp.float32),
        ],
        compiler_params=pltpu.CompilerParams(
            vmem_limit_bytes=63 * 1024 * 1024),
    )(*args)


def _bn_dice_pair(xe, xo, g, be, al):
    cnt = float(2 * xe.shape[0])
    m = (jnp.sum(xe, axis=0, keepdims=True)
         + jnp.sum(xo, axis=0, keepdims=True)) / cnt
    xce = xe - m
    xco = xo - m
    v = (jnp.sum(xce * xce, axis=0, keepdims=True)
         + jnp.sum(xco * xco, axis=0, keepdims=True)) / cnt
    rs = lax.rsqrt(v + _EPS)
    v2 = g * g * v * (rs * rs)
    s2 = lax.rsqrt(v2 + _EPS)
    gs2 = g * s2

    def dice(xc):
        xn = xc * rs
        bn = g * xn + be
        pgate = jax.nn.sigmoid(gs2 * xn)
        return bn * (al + pgate * (1.0 - al))

    return dice(xce), dice(xco)


def _tower_body(sp_ref, pn_ref, att_ref, psum_ref,
                wu1a_ref, wu1b_ref, bu1_ref, gu1_ref, beu1_ref, alu1_ref,
                wu2e_ref, wu2o_ref, bu2_ref, gu2_ref, beu2_ref, alu2_ref,
                y_ref):
    NS = sp_ref.shape[0]
    NNEG = pn_ref.shape[0] - 1
    B2 = att_ref.shape[0]
    NH = wu1a_ref.shape[2] // 2
    u = (jnp.dot(att_ref[...], wu1b_ref[...],
                 preferred_element_type=jnp.float32) + bu1_ref[...])
    for f in range(NS):
        u += jnp.dot(sp_ref[f], wu1a_ref[f],
                     preferred_element_type=jnp.float32)
    ue = u[:, 0:NH]
    uo = u[:, NH:2 * NH]
    de, do = _bn_dice_pair(ue, uo, gu1_ref[...], beu1_ref[...],
                           alu1_ref[...])
    u2 = (jnp.dot(de, wu2e_ref[...], preferred_element_type=jnp.float32)
          + jnp.dot(do, wu2o_ref[...], preferred_element_type=jnp.float32)
          + bu2_ref[...])
    cnt = float(2 * B2)
    m = jnp.dot(jnp.sum(u2, axis=0, keepdims=True), psum_ref[...],
                preferred_element_type=jnp.float32) / cnt
    xc = u2 - m
    v = jnp.dot(jnp.sum(xc * xc, axis=0, keepdims=True), psum_ref[...],
                preferred_element_type=jnp.float32) / cnt
    rs = lax.rsqrt(v + _EPS)
    g = gu2_ref[...]
    v2 = g * g * v * (rs * rs)
    s2 = lax.rsqrt(v2 + _EPS)
    xn = xc * rs
    bn = g * xn + beu2_ref[...]
    pgate = jax.nn.sigmoid(g * s2 * xn)
    al = alu2_ref[...]
    dice2 = bn * (al + pgate * (1.0 - al))
    user = _l2n_pair(dice2)
    hm = _halfmask(user.shape)
    for q in range(1 + NNEG):
        nen = _l2n_pair(pn_ref[q])
        prod = user * nen
        ye = jnp.sum(jnp.where(hm, prod, 0.0), axis=1, keepdims=True)
        yo = jnp.sum(jnp.where(hm, 0.0, prod), axis=1, keepdims=True)
        y_ref[:, q:q + 1] = ye
        y_ref[:, 1 + NNEG + q:2 + NNEG + q] = yo


def _tower(sp3, pn3, att, psum, wu1a, wu1b, bu1, gu1, beu1, alu1,
           wu2e, wu2o, bu2, gu2, beu2, alu2):
    B2 = att.shape[0]
    NNEG = pn3.shape[0] - 1
    full = lambda a: pl.BlockSpec(a.shape, lambda: (0,) * a.ndim)
    args = (sp3, pn3, att, psum, wu1a, wu1b, bu1, gu1, beu1, alu1,
            wu2e, wu2o, bu2, gu2, beu2, alu2)
    return pl.pallas_call(
        _tower_body,
        in_specs=[full(a) for a in args],
        out_specs=pl.BlockSpec((B2, 2 * (1 + NNEG)), lambda: (0, 0)),
        out_shape=jax.ShapeDtypeStruct((B2, 2 * (1 + NNEG)), jnp.float32),
        compiler_params=pltpu.CompilerParams(
            vmem_limit_bytes=63 * 1024 * 1024),
    )(*args)


def kernel(sparse_ids, hist_ids, pos_ids, neg_ids, table_sparse, table_item,
           W_a1, b_a1, g_a1, be_a1, al_a1, W_a2, b_a2,
           W_u1, b_u1, g_u1, be_u1, al_u1,
           W_u2, b_u2, g_u2, be_u2, al_u2):
    B, NS = sparse_ids.shape
    L = hist_ids.shape[1]
    NNEG = neg_ids.shape[1]
    VS = table_sparse.shape[1]
    D = table_item.shape[1]
    B2 = B // 2

    ti_pair = _relayout_pair(table_item.T[None], 6400, 16)
    ts_view = jnp.transpose(table_sparse, (0, 2, 1))
    ts_view = lax.optimization_barrier((ts_view, ti_pair))[0]
    ts_pair = _relayout_pair(ts_view, VS, 1)

    ids_item = _remap_ids(jnp.concatenate([
        hist_ids.astype(jnp.int32).T.reshape(-1),
        pos_ids.astype(jnp.int32).reshape(-1),
        neg_ids.astype(jnp.int32).T.reshape(-1),
    ]), table_item.shape[0], 6400, 16)
    ids_sp = _remap_ids(
        (sparse_ids.astype(jnp.int32).T
         + (jnp.arange(NS, dtype=jnp.int32) * VS)[:, None]).reshape(-1),
        VS, VS, 1)

    out_hist, out_pn = _sc_gather_item(ti_pair, ids_item, B * L,
                                       B * (1 + NNEG))
    out_sp = _sc_gather(ts_pair, ids_sp, B * NS)
    hist3 = out_hist.reshape(L // 2, B, _W)
    pn3 = out_pn.reshape(1 + NNEG, B2, _W)
    sp3 = out_sp.reshape(NS, B2, _W)

    NH = W_u1.shape[1]
    wu1a3 = W_u1[:NS * D].reshape(NS, D, NH)
    wu1a = (jnp.pad(wu1a3, ((0, 0), (0, D), (0, NH)))
            + jnp.pad(wu1a3, ((0, 0), (D, 0), (NH, 0))))
    wu1b = (jnp.pad(W_u1[NS * D:], ((0, D), (0, NH)))
            + jnp.pad(W_u1[NS * D:], ((D, 0), (NH, 0))))
    wu2e = jnp.pad(W_u2, ((0, 0), (0, D)))
    wu2o = jnp.pad(W_u2, ((0, 0), (D, 0)))
    psum = jnp.tile(jnp.eye(D, dtype=jnp.float32), (2, 2))
    row = lambda a: a.reshape(1, -1)
    pair = lambda a: jnp.tile(a, 2).reshape(1, -1)

    att = _attention(hist3, pn3, W_a1, row(b_a1), row(g_a1), row(be_a1),
                     row(al_a1), W_a2.reshape(1, -1), b_a2.reshape(1, 1))
    ypair = _tower(sp3, pn3, att, psum, wu1a, wu1b,
                   pair(b_u1), row(g_u1), row(be_u1), row(al_u1),
                   wu2e, wu2o, pair(b_u2), pair(g_u2), pair(be_u2),
                   pair(al_u2))
    return ypair.reshape(B, 1 + NNEG)


# speedup vs baseline: 1.3055x; 1.0285x over previous
import functools

import jax
import jax.numpy as jnp
from jax import lax
from jax.experimental import pallas as pl
from jax.experimental.pallas import tpu as pltpu
from jax.experimental.pallas import tpu_sc as plsc

_EPS = 1e-5
_CH = 128
_NW = 32
_D = 64
_W = 128


def _relayout_pair(tview, vch, nch):
    F, D, V = tview.shape
    hch = vch // 2

    def body(in_ref, out_ref):
        xt = in_ref[0].T
        out_ref[0, :, 0:D] = xt[0:hch, :]
        out_ref[0, :, D:2 * D] = xt[hch:vch, :]

    out = pl.pallas_call(
        body,
        grid=(F, nch),
        in_specs=[pl.BlockSpec((1, D, vch), lambda f, c: (f, 0, c))],
        out_specs=pl.BlockSpec((1, hch, _W), lambda f, c: (f * nch + c, 0, 0)),
        out_shape=jax.ShapeDtypeStruct((F * nch, hch, _W), jnp.float32),
    )(tview)
    return out.reshape(F * nch * vch, _D)


def _remap_ids(ids, V, vch, nch):
    f = ids // V
    vl = ids - f * V
    c = vl // vch
    local = vl - c * vch
    hch = vch // 2
    return ((f * nch + c) * vch
            + jnp.where(local < hch, 2 * local, 2 * (local - hch) + 1))


def _sc_gather_item(table, ids, n_hist, n_pn):
    nch_h = n_hist // _CH
    nch_all = (n_hist + n_pn) // _CH
    iters = -(-nch_all // _NW)
    mesh = plsc.VectorSubcoreMesh(core_axis_name="c", subcore_axis_name="s")

    @functools.partial(
        pl.kernel,
        out_type=(jax.ShapeDtypeStruct((n_hist, _D), jnp.float32),
                  jax.ShapeDtypeStruct((n_pn, _D), jnp.float32)),
        mesh=mesh,
        scratch_types=[
            pltpu.VMEM((_CH,), jnp.int32),
            pltpu.VMEM((_CH, _D), jnp.float32),
            pltpu.SemaphoreType.DMA,
        ],
        compiler_params=pltpu.CompilerParams(use_tc_tiling_on_sc=False),
    )
    def gather(tbl, ids_hbm, out_h, out_pn, idx_v, rows_v, sem):
        wid = lax.axis_index("s") * 2 + lax.axis_index("c")

        def body(i, carry):
            c = wid + _NW * i

            @pl.when(c < nch_all)
            def _():
                pltpu.sync_copy(ids_hbm.at[pl.ds(c * _CH, _CH)], idx_v)
                pltpu.async_copy(tbl.at[idx_v], rows_v, sem).wait()

                @pl.when(c < nch_h)
                def _():
                    pltpu.sync_copy(rows_v, out_h.at[pl.ds(c * _CH, _CH)])

                @pl.when(c >= nch_h)
                def _():
                    pltpu.sync_copy(
                        rows_v, out_pn.at[pl.ds((c - nch_h) * _CH, _CH)])

            return carry

        lax.fori_loop(0, iters, body, 0)

    return gather(table, ids)


def _sc_gather(table, ids, n_out):
    nchunks = n_out // _CH
    iters = -(-nchunks // _NW)
    mesh = plsc.VectorSubcoreMesh(core_axis_name="c", subcore_axis_name="s")

    @functools.partial(
        pl.kernel,
        out_type=jax.ShapeDtypeStruct((n_out, _D), jnp.float32),
        mesh=mesh,
        scratch_types=[
            pltpu.VMEM((_CH,), jnp.int32),
            pltpu.VMEM((_CH, _D), jnp.float32),
            pltpu.SemaphoreType.DMA,
        ],
        compiler_params=pltpu.CompilerParams(use_tc_tiling_on_sc=False),
    )
    def gather(tbl, ids_hbm, out, idx_v, rows_v, sem):
        wid = lax.axis_index("s") * 2 + lax.axis_index("c")

        def body(i, carry):
            c = wid + _NW * i

            @pl.when(c < nchunks)
            def _():
                base = c * _CH
                pltpu.sync_copy(ids_hbm.at[pl.ds(base, _CH)], idx_v)
                pltpu.async_copy(tbl.at[idx_v], rows_v, sem).wait()
                pltpu.sync_copy(rows_v, out.at[pl.ds(base, _CH)])

            return carry

        lax.fori_loop(0, iters, body, 0)

    return gather(table, ids)


def _halfmask(shape):
    return lax.broadcasted_iota(jnp.int32, shape, len(shape) - 1) < _D


def _l2n_pair(x):
    m = _halfmask(x.shape)
    xe = jnp.where(m, x, 0.0)
    xo = jnp.where(m, 0.0, x)
    ne = jnp.sqrt(jnp.sum(xe * xe, axis=1, keepdims=True))
    no = jnp.sqrt(jnp.sum(xo * xo, axis=1, keepdims=True))
    rcp = jnp.where(m, 1.0 / jnp.maximum(ne, 1e-12),
                    1.0 / jnp.maximum(no, 1e-12))
    return x * rcp


def _att_body(hist_ref, pn_ref,
              wa1_ref, ba1_ref, ga1_ref, bea1_ref, ala1_ref,
              wa2_ref, ba2_ref,
              att_ref,
              tgt_scr, t13e_scr, t13o_scr,
              w23a_scr, w23b_scr, w4a_scr, w4b_scr,
              he_scr, ho_scr, stat_scr, bnc_scr):
    p = pl.program_id(0)
    l = pl.program_id(1)
    L2, R, W = hist_ref.shape
    B2 = pn_ref.shape[1]
    NG = R // B2
    cnt = float(2 * R * L2)

    @pl.when((p == 0) & (l == 0))
    def _init():
        tgtp = _l2n_pair(pn_ref[0])
        for gi in range(NG):
            tgt_scr[gi * B2:(gi + 1) * B2, :] = tgtp
        z = jnp.zeros((_D, wa1_ref.shape[1]), jnp.float32)
        w13 = wa1_ref[0:_D, :] + wa1_ref[2 * _D:3 * _D, :]
        w23 = wa1_ref[_D:2 * _D, :] - wa1_ref[2 * _D:3 * _D, :]
        w4 = wa1_ref[3 * _D:4 * _D, :]
        w23a_scr[...] = jnp.concatenate([w23, z], axis=0)
        w23b_scr[...] = jnp.concatenate([z, w23], axis=0)
        w4a_scr[...] = jnp.concatenate([w4, z], axis=0)
        w4b_scr[...] = jnp.concatenate([z, w4], axis=0)
        w13a = jnp.concatenate([w13, z], axis=0)
        w13b = jnp.concatenate([z, w13], axis=0)
        te = (jnp.dot(tgtp, w13a, preferred_element_type=jnp.float32)
              + ba1_ref[...])
        to = (jnp.dot(tgtp, w13b, preferred_element_type=jnp.float32)
              + ba1_ref[...])
        for gi in range(NG):
            t13e_scr[gi * B2:(gi + 1) * B2, :] = te
            t13o_scr[gi * B2:(gi + 1) * B2, :] = to
        stat_scr[...] = jnp.zeros_like(stat_scr)

    @pl.when(p == 0)
    def _phase0():
        hl = hist_ref[l]
        thl = tgt_scr[...] * hl
        he = (jnp.dot(hl, w23a_scr[...], preferred_element_type=jnp.float32)
              + jnp.dot(thl, w4a_scr[...],
                        preferred_element_type=jnp.float32)
              + t13e_scr[...])
        ho = (jnp.dot(hl, w23b_scr[...], preferred_element_type=jnp.float32)
              + jnp.dot(thl, w4b_scr[...],
                        preferred_element_type=jnp.float32)
              + t13o_scr[...])
        he_scr[l] = he
        ho_scr[l] = ho
        stat_scr[0:1, :] += (jnp.sum(he, axis=0, keepdims=True)
                             + jnp.sum(ho, axis=0, keepdims=True))
        stat_scr[1:2, :] += (jnp.sum(he * he, axis=0, keepdims=True)
                             + jnp.sum(ho * ho, axis=0, keepdims=True))

    @pl.when((p == 1) & (l == 0))
    def _stats():
        m = stat_scr[0:1, :] / cnt
        ex2 = stat_scr[1:2, :] / cnt
        v = ex2 - m * m
        rs = lax.rsqrt(v + _EPS)
        g = ga1_ref[...]
        v2 = g * g * v * (rs * rs)
        s2 = lax.rsqrt(v2 + _EPS)
        bnc_scr[0:1, :] = m
        bnc_scr[1:2, :] = rs
        bnc_scr[2:3, :] = g * s2
        att_ref[...] = jnp.zeros_like(att_ref)

    def _dice_w(h):
        xn = (h - bnc_scr[0:1, :]) * bnc_scr[1:2, :]
        bn = ga1_ref[...] * xn + bea1_ref[...]
        pgate = jax.nn.sigmoid(bnc_scr[2:3, :] * xn)
        al = ala1_ref[...]
        dice = bn * (al + pgate * (1.0 - al))
        return (jnp.sum(dice * wa2_ref[...], axis=1, keepdims=True)
                + ba2_ref[0, 0])

    @pl.when(p == 1)
    def _phase1():
        wle = _dice_w(he_scr[l])
        wlo = _dice_w(ho_scr[l])
        hl = hist_ref[l]
        v = jnp.where(_halfmask(hl.shape), wle, wlo) * hl
        acc = v[0:B2, :]
        for gi in range(1, NG):
            acc = acc + v[gi * B2:(gi + 1) * B2, :]
        att_ref[...] += acc


def _attention(hist3, pn3, wa1, ba1, ga1, bea1, ala1, wa2, ba2):
    L2, R, W = hist3.shape
    B2 = pn3.shape[1]
    NA = wa1.shape[1]
    full = lambda a: pl.BlockSpec(a.shape, lambda p, l: (0,) * a.ndim)
    args = (hist3, pn3, wa1, ba1, ga1, bea1, ala1, wa2, ba2)
    return pl.pallas_call(
        _att_body,
        grid=(2, L2),
        in_specs=[full(a) for a in args],
        out_specs=pl.BlockSpec((B2, W), lambda p, l: (0, 0)),
        out_shape=jax.ShapeDtypeStruct((B2, W), jnp.float32),
        scratch_shapes=[
            pltpu.VMEM((R, W), jnp.float32),
            pltpu.VMEM((R, NA), jnp.float32),
            pltpu.VMEM((R, NA), jnp.float32),
            pltpu.VMEM((W, NA), jnp.float32),
            pltpu.VMEM((W, NA), jnp.float32),
            pltpu.VMEM((W, NA), jnp.float32),
            pltpu.VMEM((W, NA), jnp.float32),
            pltpu.VMEM((L2, R, NA), jnp.float32),
            pltpu.VMEM((L2, R, NA), jnp.float32),
            pltpu.VMEM((2, NA), jnp.float32),
            pltpu.VMEM((3, NA), jnp.float32),
        ],
        compiler_params=pltpu.CompilerParams(
            vmem_limit_bytes=63 * 1024 * 1024),
    )(*args)


def _bn_dice_pair(xe, xo, g, be, al):
    cnt = float(2 * xe.shape[0])
    m = (jnp.sum(xe, axis=0, keepdims=True)
         + jnp.sum(xo, axis=0, keepdims=True)) / cnt
    xce = xe - m
    xco = xo - m
    v = (jnp.sum(xce * xce, axis=0, keepdims=True)
         + jnp.sum(xco * xco, axis=0, keepdims=True)) / cnt
    rs = lax.rsqrt(v + _EPS)
    v2 = g * g * v * (rs * rs)
    s2 = lax.rsqrt(v2 + _EPS)
    gs2 = g * s2

    def dice(xc):
        xn = xc * rs
        bn = g * xn + be
        pgate = jax.nn.sigmoid(gs2 * xn)
        return bn * (al + pgate * (1.0 - al))

    return dice(xce), dice(xco)


def _tower_body(sp_ref, pn_ref, att_ref, psum_ref,
                wu1a_ref, wu1b_ref, bu1_ref, gu1_ref, beu1_ref, alu1_ref,
                wu2e_ref, wu2o_ref, bu2_ref, gu2_ref, beu2_ref, alu2_ref,
                y_ref):
    NS = sp_ref.shape[0]
    NNEG = pn_ref.shape[0] - 1
    B2 = att_ref.shape[0]
    NH = wu1a_ref.shape[2] // 2
    u = (jnp.dot(att_ref[...], wu1b_ref[...],
                 preferred_element_type=jnp.float32) + bu1_ref[...])
    for f in range(NS):
        u += jnp.dot(sp_ref[f], wu1a_ref[f],
                     preferred_element_type=jnp.float32)
    ue = u[:, 0:NH]
    uo = u[:, NH:2 * NH]
    de, do = _bn_dice_pair(ue, uo, gu1_ref[...], beu1_ref[...],
                           alu1_ref[...])
    u2 = (jnp.dot(de, wu2e_ref[...], preferred_element_type=jnp.float32)
          + jnp.dot(do, wu2o_ref[...], preferred_element_type=jnp.float32)
          + bu2_ref[...])
    cnt = float(2 * B2)
    m = jnp.dot(jnp.sum(u2, axis=0, keepdims=True), psum_ref[...],
                preferred_element_type=jnp.float32) / cnt
    xc = u2 - m
    v = jnp.dot(jnp.sum(xc * xc, axis=0, keepdims=True), psum_ref[...],
                preferred_element_type=jnp.float32) / cnt
    rs = lax.rsqrt(v + _EPS)
    g = gu2_ref[...]
    v2 = g * g * v * (rs * rs)
    s2 = lax.rsqrt(v2 + _EPS)
    xn = xc * rs
    bn = g * xn + beu2_ref[...]
    pgate = jax.nn.sigmoid(g * s2 * xn)
    al = alu2_ref[...]
    dice2 = bn * (al + pgate * (1.0 - al))
    user = _l2n_pair(dice2)
    hm = _halfmask(user.shape)
    for q in range(1 + NNEG):
        nen = _l2n_pair(pn_ref[q])
        prod = user * nen
        ye = jnp.sum(jnp.where(hm, prod, 0.0), axis=1, keepdims=True)
        yo = jnp.sum(jnp.where(hm, 0.0, prod), axis=1, keepdims=True)
        y_ref[:, q:q + 1] = ye
        y_ref[:, 1 + NNEG + q:2 + NNEG + q] = yo


def _tower(sp3, pn3, att, psum, wu1a, wu1b, bu1, gu1, beu1, alu1,
           wu2e, wu2o, bu2, gu2, beu2, alu2):
    B2 = att.shape[0]
    NNEG = pn3.shape[0] - 1
    full = lambda a: pl.BlockSpec(a.shape, lambda: (0,) * a.ndim)
    args = (sp3, pn3, att, psum, wu1a, wu1b, bu1, gu1, beu1, alu1,
            wu2e, wu2o, bu2, gu2, beu2, alu2)
    return pl.pallas_call(
        _tower_body,
        in_specs=[full(a) for a in args],
        out_specs=pl.BlockSpec((B2, 2 * (1 + NNEG)), lambda: (0, 0)),
        out_shape=jax.ShapeDtypeStruct((B2, 2 * (1 + NNEG)), jnp.float32),
        compiler_params=pltpu.CompilerParams(
            vmem_limit_bytes=63 * 1024 * 1024),
    )(*args)


def kernel(sparse_ids, hist_ids, pos_ids, neg_ids, table_sparse, table_item,
           W_a1, b_a1, g_a1, be_a1, al_a1, W_a2, b_a2,
           W_u1, b_u1, g_u1, be_u1, al_u1,
           W_u2, b_u2, g_u2, be_u2, al_u2):
    B, NS = sparse_ids.shape
    L = hist_ids.shape[1]
    NNEG = neg_ids.shape[1]
    VS = table_sparse.shape[1]
    D = table_item.shape[1]
    B2 = B // 2

    ti_pair = _relayout_pair(table_item.T[None], 6400, 16)
    ts_view = jnp.transpose(table_sparse, (0, 2, 1))
    ts_view = lax.optimization_barrier((ts_view, ti_pair))[0]
    ts_pair = _relayout_pair(ts_view, VS, 1)

    ids_item = _remap_ids(jnp.concatenate([
        hist_ids.astype(jnp.int32).T.reshape(-1),
        pos_ids.astype(jnp.int32).reshape(-1),
        neg_ids.astype(jnp.int32).T.reshape(-1),
    ]), table_item.shape[0], 6400, 16)
    ids_sp = _remap_ids(
        (sparse_ids.astype(jnp.int32).T
         + (jnp.arange(NS, dtype=jnp.int32) * VS)[:, None]).reshape(-1),
        VS, VS, 1)

    out_hist, out_pn = _sc_gather_item(ti_pair, ids_item, B * L,
                                       B * (1 + NNEG))
    out_sp = _sc_gather(ts_pair, ids_sp, B * NS)
    hist3 = out_hist.reshape(L // 5, 5 * B2, _W)
    pn3 = out_pn.reshape(1 + NNEG, B2, _W)
    sp3 = out_sp.reshape(NS, B2, _W)

    NH = W_u1.shape[1]
    wu1a3 = W_u1[:NS * D].reshape(NS, D, NH)
    wu1a = (jnp.pad(wu1a3, ((0, 0), (0, D), (0, NH)))
            + jnp.pad(wu1a3, ((0, 0), (D, 0), (NH, 0))))
    wu1b = (jnp.pad(W_u1[NS * D:], ((0, D), (0, NH)))
            + jnp.pad(W_u1[NS * D:], ((D, 0), (NH, 0))))
    wu2e = jnp.pad(W_u2, ((0, 0), (0, D)))
    wu2o = jnp.pad(W_u2, ((0, 0), (D, 0)))
    psum = jnp.tile(jnp.eye(D, dtype=jnp.float32), (2, 2))
    row = lambda a: a.reshape(1, -1)
    pair = lambda a: jnp.tile(a, 2).reshape(1, -1)

    att = _attention(hist3, pn3, W_a1, row(b_a1), row(g_a1), row(be_a1),
                     row(al_a1), W_a2.reshape(1, -1), b_a2.reshape(1, 1))
    ypair = _tower(sp3, pn3, att, psum, wu1a, wu1b,
                   pair(b_u1), row(g_u1), row(be_u1), row(al_u1),
                   wu2e, wu2o, pair(b_u2), pair(g_u2), pair(be_u2),
                   pair(al_u2))
    return ypair.reshape(B, 1 + NNEG)


# speedup vs baseline: 1.3125x; 1.0054x over previous
import functools

import jax
import jax.numpy as jnp
from jax import lax
from jax.experimental import pallas as pl
from jax.experimental.pallas import tpu as pltpu
from jax.experimental.pallas import tpu_sc as plsc

_EPS = 1e-5
_CH = 128
_NW = 32
_D = 64
_W = 128


def _relayout_pair(tview, vch, nch):
    F, D, V = tview.shape
    hch = vch // 2

    def body(in_ref, out_ref):
        xt = in_ref[0].T
        out_ref[0, :, 0:D] = xt[0:hch, :]
        out_ref[0, :, D:2 * D] = xt[hch:vch, :]

    out = pl.pallas_call(
        body,
        grid=(F, nch),
        in_specs=[pl.BlockSpec((1, D, vch), lambda f, c: (f, 0, c))],
        out_specs=pl.BlockSpec((1, hch, _W), lambda f, c: (f * nch + c, 0, 0)),
        out_shape=jax.ShapeDtypeStruct((F * nch, hch, _W), jnp.float32),
    )(tview)
    return out.reshape(F * nch * vch, _D)


def _remap_ids(ids, V, vch, nch):
    f = ids // V
    vl = ids - f * V
    c = vl // vch
    local = vl - c * vch
    hch = vch // 2
    return ((f * nch + c) * vch
            + jnp.where(local < hch, 2 * local, 2 * (local - hch) + 1))


def _sc_gather_item(table, ids, n_hist, n_pn):
    nch_h = n_hist // _CH
    nch_all = (n_hist + n_pn) // _CH
    iters = -(-nch_all // _NW)
    mesh = plsc.VectorSubcoreMesh(core_axis_name="c", subcore_axis_name="s")

    @functools.partial(
        pl.kernel,
        out_type=(jax.ShapeDtypeStruct((n_hist, _D), jnp.float32),
                  jax.ShapeDtypeStruct((n_pn, _D), jnp.float32)),
        mesh=mesh,
        scratch_types=[
            pltpu.VMEM((_CH,), jnp.int32),
            pltpu.VMEM((_CH, _D), jnp.float32),
            pltpu.SemaphoreType.DMA,
        ],
        compiler_params=pltpu.CompilerParams(use_tc_tiling_on_sc=False),
    )
    def gather(tbl, ids_hbm, out_h, out_pn, idx_v, rows_v, sem):
        wid = lax.axis_index("s") * 2 + lax.axis_index("c")

        def body(i, carry):
            c = wid + _NW * i

            @pl.when(c < nch_all)
            def _():
                pltpu.sync_copy(ids_hbm.at[pl.ds(c * _CH, _CH)], idx_v)
                pltpu.async_copy(tbl.at[idx_v], rows_v, sem).wait()

                @pl.when(c < nch_h)
                def _():
                    pltpu.sync_copy(rows_v, out_h.at[pl.ds(c * _CH, _CH)])

                @pl.when(c >= nch_h)
                def _():
                    pltpu.sync_copy(
                        rows_v, out_pn.at[pl.ds((c - nch_h) * _CH, _CH)])

            return carry

        lax.fori_loop(0, iters, body, 0)

    return gather(table, ids)


def _sc_gather(table, ids, n_out):
    nchunks = n_out // _CH
    iters = -(-nchunks // _NW)
    mesh = plsc.VectorSubcoreMesh(core_axis_name="c", subcore_axis_name="s")

    @functools.partial(
        pl.kernel,
        out_type=jax.ShapeDtypeStruct((n_out, _D), jnp.float32),
        mesh=mesh,
        scratch_types=[
            pltpu.VMEM((_CH,), jnp.int32),
            pltpu.VMEM((_CH, _D), jnp.float32),
            pltpu.SemaphoreType.DMA,
        ],
        compiler_params=pltpu.CompilerParams(use_tc_tiling_on_sc=False),
    )
    def gather(tbl, ids_hbm, out, idx_v, rows_v, sem):
        wid = lax.axis_index("s") * 2 + lax.axis_index("c")

        def body(i, carry):
            c = wid + _NW * i

            @pl.when(c < nchunks)
            def _():
                base = c * _CH
                pltpu.sync_copy(ids_hbm.at[pl.ds(base, _CH)], idx_v)
                pltpu.async_copy(tbl.at[idx_v], rows_v, sem).wait()
                pltpu.sync_copy(rows_v, out.at[pl.ds(base, _CH)])

            return carry

        lax.fori_loop(0, iters, body, 0)

    return gather(table, ids)


def _halfmask(shape):
    return lax.broadcasted_iota(jnp.int32, shape, len(shape) - 1) < _D


def _l2n_pair(x):
    m = _halfmask(x.shape)
    xe = jnp.where(m, x, 0.0)
    xo = jnp.where(m, 0.0, x)
    ne = jnp.sqrt(jnp.sum(xe * xe, axis=1, keepdims=True))
    no = jnp.sqrt(jnp.sum(xo * xo, axis=1, keepdims=True))
    rcp = jnp.where(m, 1.0 / jnp.maximum(ne, 1e-12),
                    1.0 / jnp.maximum(no, 1e-12))
    return x * rcp


def _att_body(hist_ref, pn_ref,
              wa1_ref, ba1_ref, ga1_ref, bea1_ref, ala1_ref,
              wa2_ref, ba2_ref,
              att_ref,
              tgt_scr, t13e_scr, t13o_scr,
              w23a_scr, w23b_scr, w4a_scr, w4b_scr,
              he_scr, ho_scr, stat_scr, bnc_scr):
    p = pl.program_id(0)
    l = pl.program_id(1)
    L2, R, W = hist_ref.shape
    B2 = pn_ref.shape[1]
    NG = R // B2
    cnt = float(2 * R * L2)

    @pl.when((p == 0) & (l == 0))
    def _init():
        tgtp = _l2n_pair(pn_ref[0])
        for gi in range(NG):
            tgt_scr[gi * B2:(gi + 1) * B2, :] = tgtp
        z = jnp.zeros((_D, wa1_ref.shape[1]), jnp.float32)
        w13 = wa1_ref[0:_D, :] + wa1_ref[2 * _D:3 * _D, :]
        w23 = wa1_ref[_D:2 * _D, :] - wa1_ref[2 * _D:3 * _D, :]
        w4 = wa1_ref[3 * _D:4 * _D, :]
        w23a_scr[...] = jnp.concatenate([w23, z], axis=0)
        w23b_scr[...] = jnp.concatenate([z, w23], axis=0)
        w4a_scr[...] = jnp.concatenate([w4, z], axis=0)
        w4b_scr[...] = jnp.concatenate([z, w4], axis=0)
        w13a = jnp.concatenate([w13, z], axis=0)
        w13b = jnp.concatenate([z, w13], axis=0)
        te = (jnp.dot(tgtp, w13a, preferred_element_type=jnp.float32)
              + ba1_ref[...])
        to = (jnp.dot(tgtp, w13b, preferred_element_type=jnp.float32)
              + ba1_ref[...])
        for gi in range(NG):
            t13e_scr[gi * B2:(gi + 1) * B2, :] = te
            t13o_scr[gi * B2:(gi + 1) * B2, :] = to
        stat_scr[...] = jnp.zeros_like(stat_scr)

    @pl.when(p == 0)
    def _phase0():
        hl = hist_ref[l]
        thl = tgt_scr[...] * hl
        he = (jnp.dot(hl, w23a_scr[...], preferred_element_type=jnp.float32)
              + jnp.dot(thl, w4a_scr[...],
                        preferred_element_type=jnp.float32)
              + t13e_scr[...])
        ho = (jnp.dot(hl, w23b_scr[...], preferred_element_type=jnp.float32)
              + jnp.dot(thl, w4b_scr[...],
                        preferred_element_type=jnp.float32)
              + t13o_scr[...])
        he_scr[l] = he
        ho_scr[l] = ho
        stat_scr[0:1, :] += (jnp.sum(he, axis=0, keepdims=True)
                             + jnp.sum(ho, axis=0, keepdims=True))
        stat_scr[1:2, :] += (jnp.sum(he * he, axis=0, keepdims=True)
                             + jnp.sum(ho * ho, axis=0, keepdims=True))

    @pl.when((p == 1) & (l == 0))
    def _stats():
        m = stat_scr[0:1, :] / cnt
        ex2 = stat_scr[1:2, :] / cnt
        v = ex2 - m * m
        rs = lax.rsqrt(v + _EPS)
        g = ga1_ref[...]
        v2 = g * g * v * (rs * rs)
        s2 = lax.rsqrt(v2 + _EPS)
        bnc_scr[0:1, :] = m
        bnc_scr[1:2, :] = rs
        bnc_scr[2:3, :] = g * s2
        att_ref[...] = jnp.zeros_like(att_ref)

    def _dice_w(h):
        xn = (h - bnc_scr[0:1, :]) * bnc_scr[1:2, :]
        bn = ga1_ref[...] * xn + bea1_ref[...]
        pgate = jax.nn.sigmoid(bnc_scr[2:3, :] * xn)
        al = ala1_ref[...]
        dice = bn * (al + pgate * (1.0 - al))
        return (jnp.sum(dice * wa2_ref[...], axis=1, keepdims=True)
                + ba2_ref[0, 0])

    @pl.when(p == 1)
    def _phase1():
        wle = _dice_w(he_scr[l])
        wlo = _dice_w(ho_scr[l])
        hl = hist_ref[l]
        v = jnp.where(_halfmask(hl.shape), wle, wlo) * hl
        acc = v[0:B2, :]
        for gi in range(1, NG):
            acc = acc + v[gi * B2:(gi + 1) * B2, :]
        att_ref[...] += acc


def _attention(hist3, pn3, wa1, ba1, ga1, bea1, ala1, wa2, ba2):
    L2, R, W = hist3.shape
    B2 = pn3.shape[1]
    NA = wa1.shape[1]
    full = lambda a: pl.BlockSpec(a.shape, lambda p, l: (0,) * a.ndim)
    args = (hist3, pn3, wa1, ba1, ga1, bea1, ala1, wa2, ba2)
    return pl.pallas_call(
        _att_body,
        grid=(2, L2),
        in_specs=[full(a) for a in args],
        out_specs=pl.BlockSpec((B2, W), lambda p, l: (0, 0)),
        out_shape=jax.ShapeDtypeStruct((B2, W), jnp.float32),
        scratch_shapes=[
            pltpu.VMEM((R, W), jnp.float32),
            pltpu.VMEM((R, NA), jnp.float32),
            pltpu.VMEM((R, NA), jnp.float32),
            pltpu.VMEM((W, NA), jnp.float32),
            pltpu.VMEM((W, NA), jnp.float32),
            pltpu.VMEM((W, NA), jnp.float32),
            pltpu.VMEM((W, NA), jnp.float32),
            pltpu.VMEM((L2, R, NA), jnp.float32),
            pltpu.VMEM((L2, R, NA), jnp.float32),
            pltpu.VMEM((2, NA), jnp.float32),
            pltpu.VMEM((3, NA), jnp.float32),
        ],
        compiler_params=pltpu.CompilerParams(
            vmem_limit_bytes=63 * 1024 * 1024),
    )(*args)


def _bn_dice_pair(xe, xo, g, be, al):
    cnt = float(2 * xe.shape[0])
    m = (jnp.sum(xe, axis=0, keepdims=True)
         + jnp.sum(xo, axis=0, keepdims=True)) / cnt
    xce = xe - m
    xco = xo - m
    v = (jnp.sum(xce * xce, axis=0, keepdims=True)
         + jnp.sum(xco * xco, axis=0, keepdims=True)) / cnt
    rs = lax.rsqrt(v + _EPS)
    v2 = g * g * v * (rs * rs)
    s2 = lax.rsqrt(v2 + _EPS)
    gs2 = g * s2

    def dice(xc):
        xn = xc * rs
        bn = g * xn + be
        pgate = jax.nn.sigmoid(gs2 * xn)
        return bn * (al + pgate * (1.0 - al))

    return dice(xce), dice(xco)


def _tower_body(sp_ref, pn_ref, att_ref, psum_ref,
                wu1a_ref, wu1b_ref, bu1_ref, gu1_ref, beu1_ref, alu1_ref,
                wu2e_ref, wu2o_ref, bu2_ref, gu2_ref, beu2_ref, alu2_ref,
                y_ref):
    NS = sp_ref.shape[0]
    NNEG = pn_ref.shape[0] - 1
    B2 = att_ref.shape[0]
    NH = wu1a_ref.shape[2] // 2
    u = (jnp.dot(att_ref[...], wu1b_ref[...],
                 preferred_element_type=jnp.float32) + bu1_ref[...])
    for f in range(NS):
        u += jnp.dot(sp_ref[f], wu1a_ref[f],
                     preferred_element_type=jnp.float32)
    ue = u[:, 0:NH]
    uo = u[:, NH:2 * NH]
    de, do = _bn_dice_pair(ue, uo, gu1_ref[...], beu1_ref[...],
                           alu1_ref[...])
    u2 = (jnp.dot(de, wu2e_ref[...], preferred_element_type=jnp.float32)
          + jnp.dot(do, wu2o_ref[...], preferred_element_type=jnp.float32)
          + bu2_ref[...])
    cnt = float(2 * B2)
    m = jnp.dot(jnp.sum(u2, axis=0, keepdims=True), psum_ref[...],
                preferred_element_type=jnp.float32) / cnt
    xc = u2 - m
    v = jnp.dot(jnp.sum(xc * xc, axis=0, keepdims=True), psum_ref[...],
                preferred_element_type=jnp.float32) / cnt
    rs = lax.rsqrt(v + _EPS)
    g = gu2_ref[...]
    v2 = g * g * v * (rs * rs)
    s2 = lax.rsqrt(v2 + _EPS)
    xn = xc * rs
    bn = g * xn + beu2_ref[...]
    pgate = jax.nn.sigmoid(g * s2 * xn)
    al = alu2_ref[...]
    dice2 = bn * (al + pgate * (1.0 - al))
    user = _l2n_pair(dice2)
    hm = _halfmask(user.shape)
    for q in range(1 + NNEG):
        nen = _l2n_pair(pn_ref[q])
        prod = user * nen
        ye = jnp.sum(jnp.where(hm, prod, 0.0), axis=1, keepdims=True)
        yo = jnp.sum(jnp.where(hm, 0.0, prod), axis=1, keepdims=True)
        y_ref[:, q:q + 1] = ye
        y_ref[:, 1 + NNEG + q:2 + NNEG + q] = yo


def _tower(sp3, pn3, att, psum, wu1a, wu1b, bu1, gu1, beu1, alu1,
           wu2e, wu2o, bu2, gu2, beu2, alu2):
    B2 = att.shape[0]
    NNEG = pn3.shape[0] - 1
    full = lambda a: pl.BlockSpec(a.shape, lambda: (0,) * a.ndim)
    args = (sp3, pn3, att, psum, wu1a, wu1b, bu1, gu1, beu1, alu1,
            wu2e, wu2o, bu2, gu2, beu2, alu2)
    return pl.pallas_call(
        _tower_body,
        in_specs=[full(a) for a in args],
        out_specs=pl.BlockSpec((B2, 2 * (1 + NNEG)), lambda: (0, 0)),
        out_shape=jax.ShapeDtypeStruct((B2, 2 * (1 + NNEG)), jnp.float32),
        compiler_params=pltpu.CompilerParams(
            vmem_limit_bytes=63 * 1024 * 1024),
    )(*args)


def kernel(sparse_ids, hist_ids, pos_ids, neg_ids, table_sparse, table_item,
           W_a1, b_a1, g_a1, be_a1, al_a1, W_a2, b_a2,
           W_u1, b_u1, g_u1, be_u1, al_u1,
           W_u2, b_u2, g_u2, be_u2, al_u2):
    B, NS = sparse_ids.shape
    L = hist_ids.shape[1]
    NNEG = neg_ids.shape[1]
    VS = table_sparse.shape[1]
    D = table_item.shape[1]
    B2 = B // 2

    ti_pair = _relayout_pair(table_item.T[None], 6400, 16)
    ts_view = jnp.transpose(table_sparse, (0, 2, 1))
    ts_view = lax.optimization_barrier((ts_view, ti_pair))[0]
    ts_pair = _relayout_pair(ts_view, VS, 1)

    ids_item = _remap_ids(jnp.concatenate([
        hist_ids.astype(jnp.int32).T.reshape(-1),
        pos_ids.astype(jnp.int32).reshape(-1),
        neg_ids.astype(jnp.int32).T.reshape(-1),
    ]), table_item.shape[0], 6400, 16)
    ids_sp = _remap_ids(
        (sparse_ids.astype(jnp.int32).T
         + (jnp.arange(NS, dtype=jnp.int32) * VS)[:, None]).reshape(-1),
        VS, VS, 1)

    out_hist, out_pn = _sc_gather_item(ti_pair, ids_item, B * L,
                                       B * (1 + NNEG))
    out_sp = _sc_gather(ts_pair, ids_sp, B * NS)
    hist3 = out_hist.reshape(L // 10, 10 * B2, _W)
    pn3 = out_pn.reshape(1 + NNEG, B2, _W)
    sp3 = out_sp.reshape(NS, B2, _W)

    NH = W_u1.shape[1]
    wu1a3 = W_u1[:NS * D].reshape(NS, D, NH)
    wu1a = (jnp.pad(wu1a3, ((0, 0), (0, D), (0, NH)))
            + jnp.pad(wu1a3, ((0, 0), (D, 0), (NH, 0))))
    wu1b = (jnp.pad(W_u1[NS * D:], ((0, D), (0, NH)))
            + jnp.pad(W_u1[NS * D:], ((D, 0), (NH, 0))))
    wu2e = jnp.pad(W_u2, ((0, 0), (0, D)))
    wu2o = jnp.pad(W_u2, ((0, 0), (D, 0)))
    psum = jnp.tile(jnp.eye(D, dtype=jnp.float32), (2, 2))
    row = lambda a: a.reshape(1, -1)
    pair = lambda a: jnp.tile(a, 2).reshape(1, -1)

    att = _attention(hist3, pn3, W_a1, row(b_a1), row(g_a1), row(be_a1),
                     row(al_a1), W_a2.reshape(1, -1), b_a2.reshape(1, 1))
    ypair = _tower(sp3, pn3, att, psum, wu1a, wu1b,
                   pair(b_u1), row(g_u1), row(be_u1), row(al_u1),
                   wu2e, wu2o, pair(b_u2), pair(g_u2), pair(be_u2),
                   pair(al_u2))
    return ypair.reshape(B, 1 + NNEG)
